# Initial kernel scaffold; baseline (speedup 1.0000x reference)
import jax
import jax.numpy as jnp
from jax.experimental import pallas as pl


def kernel(x, adj, W1, b1, W2, b2):
    raise NotImplementedError("write your pallas kernel here")


# speedup vs baseline: 5323.0218x; 5323.0218x over previous
import functools

import jax
import jax.numpy as jnp
from jax.experimental import pallas as pl

_B, _N, _F = 4, 512, 128


def _gcn_batch_kernel(adj_ref, x_ref, w1_ref, b1_ref, w2_ref, b2_ref, out_ref):
    a = adj_ref[0]
    r = jax.lax.broadcasted_iota(jnp.int32, a.shape, 0)
    c = jax.lax.broadcasted_iota(jnp.int32, a.shape, 1)
    a = a + (r == c).astype(a.dtype)

    ones = jnp.ones((a.shape[0], 1), a.dtype)
    dn = (((0,), (0,)), ((), ()))
    deg = jax.lax.dot_general(a, ones, dn, preferred_element_type=jnp.float32)
    dinv = jax.lax.rsqrt(deg)

    xw = jnp.dot(x_ref[0], w1_ref[:], preferred_element_type=jnp.float32)
    t = jax.lax.dot_general(a, xw * dinv, dn, preferred_element_type=jnp.float32)
    h = jnp.maximum(t * dinv + b1_ref[:], 0.0)

    hw = jnp.dot(h, w2_ref[:], preferred_element_type=jnp.float32)
    t2 = jax.lax.dot_general(a, hw * dinv, dn, preferred_element_type=jnp.float32)
    h2 = t2 * dinv + b2_ref[:]

    out_ref[0] = jnp.sum(h2, axis=0, keepdims=True) * (1.0 / h2.shape[0])


@jax.jit
def kernel(x, adj, W1, b1, W2, b2):
    b1r = b1.reshape(1, -1)
    b2r = b2.reshape(1, -1)
    grid = (_B,)
    return pl.pallas_call(
        _gcn_batch_kernel,
        grid=grid,
        in_specs=[
            pl.BlockSpec((1, _N, _N), lambda b: (b, 0, 0)),
            pl.BlockSpec((1, _N, _F), lambda b: (b, 0, 0)),
            pl.BlockSpec((_F, _F), lambda b: (0, 0)),
            pl.BlockSpec((1, _F), lambda b: (0, 0)),
            pl.BlockSpec((_F, _F), lambda b: (0, 0)),
            pl.BlockSpec((1, _F), lambda b: (0, 0)),
        ],
        out_specs=pl.BlockSpec((1, 1, _F), lambda b: (b, 0, 0)),
        out_shape=jax.ShapeDtypeStruct((_B, 1, _F), jnp.float32),
    )(adj, x, W1, b1r, W2, b2r).reshape(_B, _F)
